# baseline (speedup 1.0000x reference)
import jax
import jax.numpy as jnp
from jax import lax
from jax.experimental import pallas as pl
from jax.experimental.pallas import tpu as pltpu
from jax.experimental.pallas import tpu_sc as plsc

_MODAL_SHAPE = (4095, 2048)
_T = _MODAL_SHAPE[0] * _MODAL_SHAPE[1]
_N = 4193280

_NW = 32
_B = 32768
_NB = 256
_B_LAST = _T - (_NB - 1) * _B
_SUB = _NB // _NW
_C = 4096
_W = _C + 16
_SENT = 1 << 23
_SEARCH_ROUNDS = 23


def _sc_scatter_body(idx_hbm, oro_hbm, idx_tail_hbm, oro_tail_hbm, out_hbm,
                     sbuf, probe, idx_b0, idx_b1, val_b0, val_b1, out_block,
                     sem_i0, sem_v0, sem_i1, sem_v1):
    wid = lax.axis_index("s") * 2 + lax.axis_index("c")
    iota = lax.iota(jnp.int32, 16)

    targets = (wid * _SUB + iota) * _B
    lo_v = jnp.zeros((16,), jnp.int32)
    hi_v = jnp.full((16,), _N, jnp.int32)
    for _ in range(_SEARCH_ROUNDS):
        mid = (lo_v + hi_v) // 2
        pltpu.sync_copy(idx_hbm.at[jnp.minimum(mid, _N - 1)], probe)
        v = probe[pl.ds(0, 16)]
        active = lo_v < hi_v
        c = v < targets
        lo_v = jnp.where(active & c, mid + 1, lo_v)
        hi_v = jnp.where(active & (~c), mid, hi_v)
    sbuf[pl.ds(0, 16)] = lo_v

    zero16 = jnp.zeros((16,), jnp.float32)
    bufs = ((idx_b0, val_b0, sem_i0, sem_v0), (idx_b1, val_b1, sem_i1, sem_v1))

    def process_block(kb, _):
        blk = wid * _SUB + kb
        lo = blk * _B
        hi = lo + _B

        s = jnp.max(plsc.load_gather(sbuf, [jnp.full((16,), kb, jnp.int32)]))
        e = jnp.max(plsc.load_gather(sbuf, [jnp.full((16,), kb + 1,
                                                     jnp.int32)]))

        def zero_body(i, _):
            base = i * 512
            for u in range(32):
                out_block[pl.ds(base + u * 16, 16)] = zero16
            return 0
        lax.fori_loop(0, _B // 512, zero_body, 0)

        s_al = (s // 16) * 16
        nt = (e - s_al + _C - 1) // _C

        def issue(t, b):
            ibuf, vbuf, isem, vsem = bufs[b]
            g = s_al + t * _C

            @pl.when(g <= _N - _W)
            def _():
                pltpu.async_copy(idx_hbm.at[pl.ds(g, _W)], ibuf, isem)
                pltpu.async_copy(oro_hbm.at[pl.ds(g, _W)], vbuf, vsem)

            @pl.when(g > _N - _W)
            def _():
                d = g - (_N - _W)
                pltpu.async_copy(idx_tail_hbm.at[pl.ds(d, _W)], ibuf, isem)
                pltpu.async_copy(oro_tail_hbm.at[pl.ds(d, _W)], vbuf, vsem)

        def wait(b):
            ibuf, vbuf, isem, vsem = bufs[b]
            pltpu.make_async_copy(idx_hbm.at[pl.ds(0, _W)], ibuf, isem).wait()
            pltpu.make_async_copy(oro_hbm.at[pl.ds(0, _W)], vbuf, vsem).wait()

        def process(b):
            ibuf, vbuf, _, _ = bufs[b]
            for k in range(_C // 16):
                base = k * 16
                idx_v = ibuf[pl.ds(base, 16)]
                val_v = vbuf[pl.ds(base, 16)]
                nxt_v = plsc.load_gather(ibuf, [iota + (base + 1)])
                m = (idx_v >= lo) & (idx_v < hi) & (idx_v != nxt_v)
                plsc.store_scatter(out_block, [idx_v - lo], val_v, mask=m)

        @pl.when(nt > 0)
        def _():
            issue(0, 0)

        def pair_body(t2, _):
            t0 = 2 * t2

            @pl.when(t0 + 1 < nt)
            def _():
                issue(t0 + 1, 1)

            wait(0)
            process(0)

            @pl.when(t0 + 2 < nt)
            def _():
                issue(t0 + 2, 0)

            @pl.when(t0 + 1 < nt)
            def _():
                wait(1)
                process(1)
            return 0
        lax.fori_loop(0, (nt + 1) // 2, pair_body, 0)

        @pl.when(blk < _NB - 1)
        def _():
            pltpu.sync_copy(out_block, out_hbm.at[pl.ds(lo, _B)])

        @pl.when(blk == _NB - 1)
        def _():
            pltpu.sync_copy(out_block.at[pl.ds(0, _B_LAST)],
                            out_hbm.at[pl.ds(lo, _B_LAST)])
        return 0

    lax.fori_loop(0, _SUB, process_block, 0)


@jax.jit
def _modal_scatter(orography, mask_idx):
    idx32 = mask_idx.astype(jnp.int32)
    idx_tail = jnp.concatenate(
        [idx32[_N - _W:], jnp.full((_W,), _SENT, jnp.int32)])
    oro_tail = jnp.concatenate(
        [orography[_N - _W:], jnp.zeros((_W,), jnp.float32)])

    mesh = plsc.VectorSubcoreMesh(core_axis_name="c", subcore_axis_name="s")
    flat = pl.kernel(
        _sc_scatter_body,
        out_type=jax.ShapeDtypeStruct((_T,), jnp.float32),
        mesh=mesh,
        compiler_params=pltpu.CompilerParams(needs_layout_passes=False),
        scratch_types=[
            pltpu.VMEM((16,), jnp.int32),
            pltpu.VMEM((16,), jnp.int32),
            pltpu.VMEM((_W,), jnp.int32),
            pltpu.VMEM((_W,), jnp.int32),
            pltpu.VMEM((_W,), jnp.float32),
            pltpu.VMEM((_W,), jnp.float32),
            pltpu.VMEM((_B,), jnp.float32),
            pltpu.SemaphoreType.DMA,
            pltpu.SemaphoreType.DMA,
            pltpu.SemaphoreType.DMA,
            pltpu.SemaphoreType.DMA,
        ],
    )(idx32, orography, idx_tail, oro_tail)
    return flat.reshape(_MODAL_SHAPE)


def kernel(orography, mask_idx):
    return _modal_scatter(orography, mask_idx)


# speedup vs baseline: 1.4788x; 1.4788x over previous
import jax
import jax.numpy as jnp
from jax import lax
from jax.experimental import pallas as pl
from jax.experimental.pallas import tpu as pltpu
from jax.experimental.pallas import tpu_sc as plsc

_MODAL_SHAPE = (4095, 2048)
_T = _MODAL_SHAPE[0] * _MODAL_SHAPE[1]
_N = 4193280

_NW = 32
_B = 32768
_NB = 256
_B_LAST = _T - (_NB - 1) * _B
_SUB = _NB // _NW
_C = 1024
_W = _C + 16
_SENT = 1 << 23
_SEARCH_ROUNDS = 23


def _sc_scatter_body(idx_hbm, oro_hbm, idx_tail_hbm, oro_tail_hbm, out_hbm,
                     sbuf, probe, idx_b0, idx_b1, val_b0, val_b1, out_block,
                     sem_i0, sem_v0, sem_i1, sem_v1):
    wid = lax.axis_index("s") * 2 + lax.axis_index("c")
    iota = lax.iota(jnp.int32, 16)

    targets = (wid * _SUB + iota) * _B
    lo_v = jnp.zeros((16,), jnp.int32)
    hi_v = jnp.full((16,), _N, jnp.int32)
    for _ in range(_SEARCH_ROUNDS):
        mid = (lo_v + hi_v) // 2
        pltpu.sync_copy(idx_hbm.at[jnp.minimum(mid, _N - 1)], probe)
        v = probe[pl.ds(0, 16)]
        active = lo_v < hi_v
        c = v < targets
        lo_v = jnp.where(active & c, mid + 1, lo_v)
        hi_v = jnp.where(active & (~c), mid, hi_v)
    sbuf[pl.ds(0, 16)] = lo_v

    zero16 = jnp.zeros((16,), jnp.float32)
    bufs = ((idx_b0, val_b0, sem_i0, sem_v0), (idx_b1, val_b1, sem_i1, sem_v1))

    def process_block(kb, _):
        blk = wid * _SUB + kb
        lo = blk * _B
        hi = lo + _B

        s = jnp.max(plsc.load_gather(sbuf, [jnp.full((16,), kb, jnp.int32)]))
        e = jnp.max(plsc.load_gather(sbuf, [jnp.full((16,), kb + 1,
                                                     jnp.int32)]))

        def zero_body(i, _):
            base = i * 512
            for u in range(32):
                out_block[pl.ds(base + u * 16, 16)] = zero16
            return 0
        lax.fori_loop(0, _B // 512, zero_body, 0)

        s_al = (s // 16) * 16
        nt = (e - s_al + _C - 1) // _C

        def issue(t, b):
            ibuf, vbuf, isem, vsem = bufs[b]
            g = s_al + t * _C

            @pl.when(g <= _N - _W)
            def _():
                pltpu.async_copy(idx_hbm.at[pl.ds(g, _W)], ibuf, isem)
                pltpu.async_copy(oro_hbm.at[pl.ds(g, _W)], vbuf, vsem)

            @pl.when(g > _N - _W)
            def _():
                d = g - (_N - _W)
                pltpu.async_copy(idx_tail_hbm.at[pl.ds(d, _W)], ibuf, isem)
                pltpu.async_copy(oro_tail_hbm.at[pl.ds(d, _W)], vbuf, vsem)

        def wait(b):
            ibuf, vbuf, isem, vsem = bufs[b]
            pltpu.make_async_copy(idx_hbm.at[pl.ds(0, _W)], ibuf, isem).wait()
            pltpu.make_async_copy(oro_hbm.at[pl.ds(0, _W)], vbuf, vsem).wait()

        def process(b):
            ibuf, vbuf, _, _ = bufs[b]
            for k in range(_C // 16):
                base = k * 16
                idx_v = ibuf[pl.ds(base, 16)]
                val_v = vbuf[pl.ds(base, 16)]
                nxt_v = plsc.load_gather(ibuf, [iota + (base + 1)])
                m = (idx_v >= lo) & (idx_v < hi) & (idx_v != nxt_v)
                plsc.store_scatter(out_block, [idx_v - lo], val_v, mask=m)

        @pl.when(nt > 0)
        def _():
            issue(0, 0)

        def pair_body(t2, _):
            t0 = 2 * t2

            @pl.when(t0 + 1 < nt)
            def _():
                issue(t0 + 1, 1)

            wait(0)
            process(0)

            @pl.when(t0 + 2 < nt)
            def _():
                issue(t0 + 2, 0)

            @pl.when(t0 + 1 < nt)
            def _():
                wait(1)
                process(1)
            return 0
        lax.fori_loop(0, (nt + 1) // 2, pair_body, 0)

        @pl.when(blk < _NB - 1)
        def _():
            pltpu.sync_copy(out_block, out_hbm.at[pl.ds(lo, _B)])

        @pl.when(blk == _NB - 1)
        def _():
            pltpu.sync_copy(out_block.at[pl.ds(0, _B_LAST)],
                            out_hbm.at[pl.ds(lo, _B_LAST)])
        return 0

    lax.fori_loop(0, _SUB, process_block, 0)


@jax.jit
def _modal_scatter(orography, mask_idx):
    idx32 = mask_idx.astype(jnp.int32)
    idx_tail = jnp.concatenate(
        [idx32[_N - _W:], jnp.full((_W,), _SENT, jnp.int32)])
    oro_tail = jnp.concatenate(
        [orography[_N - _W:], jnp.zeros((_W,), jnp.float32)])

    mesh = plsc.VectorSubcoreMesh(core_axis_name="c", subcore_axis_name="s")
    flat = pl.kernel(
        _sc_scatter_body,
        out_type=jax.ShapeDtypeStruct((_T,), jnp.float32),
        mesh=mesh,
        compiler_params=pltpu.CompilerParams(needs_layout_passes=False),
        scratch_types=[
            pltpu.VMEM((16,), jnp.int32),
            pltpu.VMEM((16,), jnp.int32),
            pltpu.VMEM((_W,), jnp.int32),
            pltpu.VMEM((_W,), jnp.int32),
            pltpu.VMEM((_W,), jnp.float32),
            pltpu.VMEM((_W,), jnp.float32),
            pltpu.VMEM((_B,), jnp.float32),
            pltpu.SemaphoreType.DMA,
            pltpu.SemaphoreType.DMA,
            pltpu.SemaphoreType.DMA,
            pltpu.SemaphoreType.DMA,
        ],
    )(idx32, orography, idx_tail, oro_tail)
    return flat.reshape(_MODAL_SHAPE)


def kernel(orography, mask_idx):
    return _modal_scatter(orography, mask_idx)
